# Initial kernel scaffold; baseline (speedup 1.0000x reference)
import jax
import jax.numpy as jnp
from jax.experimental import pallas as pl


def kernel(input_ids, table, gamma, beta):
    raise NotImplementedError("write your pallas kernel here")


# speedup vs baseline: 20.7393x; 20.7393x over previous
import functools

import jax
import jax.numpy as jnp
from jax import lax
from jax.experimental import pallas as pl
from jax.experimental.pallas import tpu as pltpu
from jax.experimental.pallas import tpu_sc as plsc

VOCAB = 100000
D = 64
LN_EPS = 1e-12

B, F, L = 1024, 26, 50
S = B * F

NC, NS = 2, 16
NW = NC * NS
SEG_W = S // NW
CH = 16
NCHUNK = SEG_W // CH
TPG = 100
GPC = CH * L // TPG


def _ln_table_body(tab_ref, out_ref):
    x = tab_ref[...]
    mu = jnp.mean(x, axis=-1, keepdims=True)
    var = jnp.mean((x - mu) ** 2, axis=-1, keepdims=True)
    out_ref[...] = (x - mu) * lax.rsqrt(var + LN_EPS)


def _normalize_table(table):
    return pl.pallas_call(
        _ln_table_body,
        grid=(100,),
        in_specs=[pl.BlockSpec((VOCAB // 100, D), lambda i: (i, 0))],
        out_specs=pl.BlockSpec((VOCAB // 100, D), lambda i: (i, 0)),
        out_shape=jax.ShapeDtypeStruct((VOCAB, D), jnp.float32),
    )(table)


def _seg_sum_body(ids_hbm, tabn_hbm, gamma_hbm, beta_hbm, out_hbm,
                  idx_v, rows_v, out_v, gamma_v, beta_v, sem):
    wid = lax.axis_index("s") * NC + lax.axis_index("c")

    pltpu.sync_copy(gamma_hbm, gamma_v)
    pltpu.sync_copy(beta_hbm, beta_v)

    @pl.loop(0, NCHUNK)
    def _chunk(c):
        seg0 = pl.multiple_of(wid * SEG_W + c * CH, 8)
        row0 = pl.multiple_of((wid * SEG_W // 2) + c * GPC, 8)
        pltpu.sync_copy(ids_hbm.at[pl.ds(row0, GPC)], idx_v)
        copies = [
            pltpu.async_copy(
                tabn_hbm.at[idx_v.at[j]],
                rows_v.at[pl.ds(j * TPG, TPG)],
                sem,
            )
            for j in range(GPC)
        ]
        for cp in copies:
            cp.wait()

        @pl.loop(0, CH)
        def _seg(s):
            base = s * L
            for q in range(4):
                sl = pl.ds(q * 16, 16)
                acc = rows_v[base, sl]
                for l in range(1, L):
                    acc = acc + rows_v[base + l, sl]
                out_v[s, sl] = acc * gamma_v[sl] + beta_v[sl] * 50.0

        pltpu.sync_copy(out_v, out_hbm.at[pl.ds(seg0, CH)])


def _gather_sum(ids2d, table_n, gamma, beta):
    mesh = plsc.VectorSubcoreMesh(core_axis_name="c", subcore_axis_name="s")
    f = functools.partial(
        pl.kernel,
        out_type=jax.ShapeDtypeStruct((S, D), jnp.float32),
        mesh=mesh,
        compiler_params=pltpu.CompilerParams(use_tc_tiling_on_sc=False),
        scratch_types=[
            pltpu.VMEM((GPC, TPG), jnp.int32),
            pltpu.VMEM((CH * L, D), jnp.float32),
            pltpu.VMEM((CH, D), jnp.float32),
            pltpu.VMEM((D,), jnp.float32),
            pltpu.VMEM((D,), jnp.float32),
            pltpu.SemaphoreType.DMA,
        ],
    )(_seg_sum_body)
    return f(ids2d, table_n, gamma, beta)


def kernel(input_ids, table, gamma, beta):
    table_n = _normalize_table(table)
    ids2d = input_ids.reshape(S * L // TPG, TPG)
    out = _gather_sum(ids2d, table_n, gamma, beta)
    return out.reshape(B, F, D)


# speedup vs baseline: 43.6635x; 2.1054x over previous
import functools

import jax
import jax.numpy as jnp
from jax import lax
from jax.experimental import pallas as pl
from jax.experimental.pallas import tpu as pltpu
from jax.experimental.pallas import tpu_sc as plsc

VOCAB = 100000
D = 64
LN_EPS = 1e-12

B, F, L = 1024, 26, 50
S = B * F

NC, NS = 2, 16
NW = NC * NS
SEG_W = S // NW
CH = 16
NCHUNK = SEG_W // CH
TPG = 100
GPC = CH * L // TPG


def _ln_table_body(tab_ref, out_ref):
    x = tab_ref[...]
    mu = jnp.mean(x, axis=-1, keepdims=True)
    var = jnp.mean((x - mu) ** 2, axis=-1, keepdims=True)
    out_ref[...] = ((x - mu) * lax.rsqrt(var + LN_EPS)).astype(jnp.bfloat16)


def _normalize_table(table):
    return pl.pallas_call(
        _ln_table_body,
        grid=(25,),
        in_specs=[pl.BlockSpec((VOCAB // 25, D), lambda i: (i, 0))],
        out_specs=pl.BlockSpec((VOCAB // 25, D), lambda i: (i, 0)),
        out_shape=jax.ShapeDtypeStruct((VOCAB, D), jnp.bfloat16),
    )(table)


def _seg_sum_body(ids_hbm, tabn_hbm, gamma_hbm, beta_hbm, out_hbm,
                  idx_v, rows_v, out_v, gb_v, isem, gsem, osem):
    wid = lax.axis_index("s") * NC + lax.axis_index("c")
    base_seg = wid * SEG_W
    base_row = wid * (SEG_W * L // TPG)

    pltpu.sync_copy(gamma_hbm, gb_v.at[pl.ds(0, D)])
    pltpu.sync_copy(beta_hbm, gb_v.at[pl.ds(D, D)])
    iota2 = lax.iota(jnp.int32, 16) * 2
    colv = [iota2, iota2 + 1, iota2 + 32, iota2 + 33]
    gvec = [plsc.load_gather(gb_v, [colv[k]]) for k in range(4)]
    bvec = [plsc.load_gather(gb_v, [colv[k] + D]) * 50.0 for k in range(4)]

    def issue_idx(c, p):
        row0 = pl.multiple_of(base_row + c * GPC, 8)
        return pltpu.async_copy(
            ids_hbm.at[pl.ds(row0, GPC)], idx_v.at[p], isem[p])

    def wait_idx(p):
        pltpu.make_async_copy(
            ids_hbm.at[pl.ds(0, GPC)], idx_v.at[p], isem[p]).wait()

    def issue_gathers(p):
        for j in range(GPC):
            pltpu.async_copy(
                tabn_hbm.at[idx_v.at[p, j]],
                rows_v.at[p, pl.ds(j * TPG, TPG)],
                gsem[p])

    def wait_gathers(p):
        for j in range(GPC):
            pltpu.make_async_copy(
                tabn_hbm.at[idx_v.at[p, j]],
                rows_v.at[p, pl.ds(j * TPG, TPG)],
                gsem[p]).wait()

    def issue_out(c, p):
        seg0 = pl.multiple_of(base_seg + c * CH, 8)
        return pltpu.async_copy(
            out_v.at[p], out_hbm.at[pl.ds(seg0, CH)], osem[p])

    def wait_out(p):
        pltpu.make_async_copy(
            out_v.at[p], out_hbm.at[pl.ds(0, CH)], osem[p]).wait()

    def compute(p):
        @pl.loop(0, CH)
        def _seg(s):
            base = s * L
            acc = [jnp.zeros((16,), jnp.float32) for _ in range(4)]
            for t in range(L // 2):
                r0 = base + 2 * t
                for h in range(2):
                    a = rows_v[p, r0, pl.ds(h * 32, 32)]
                    b = rows_v[p, r0 + 1, pl.ds(h * 32, 32)]
                    pair = plsc.bitcast(a + b, jnp.int32)
                    lo = plsc.bitcast(pair << 16, jnp.float32)
                    hi = plsc.bitcast(pair & jnp.int32(-65536), jnp.float32)
                    acc[2 * h] = acc[2 * h] + lo
                    acc[2 * h + 1] = acc[2 * h + 1] + hi
            srow = jnp.broadcast_to(s, (16,))
            for k in range(4):
                plsc.store_scatter(
                    out_v.at[p], [srow, colv[k]], acc[k] * gvec[k] + bvec[k])

    issue_idx(0, 0)
    wait_idx(0)
    issue_gathers(0)
    issue_idx(1, 1)

    @pl.loop(0, NCHUNK, step=2)
    def _chunk(c0):
        for par in (0, 1):
            c = c0 + par

            @pl.when(c + 1 < NCHUNK)
            def _():
                wait_idx(1 - par)
                issue_gathers(1 - par)

            wait_gathers(par)

            @pl.when(c + 2 < NCHUNK)
            def _():
                issue_idx(c + 2, par)

            @pl.when(c >= 2)
            def _():
                wait_out(par)

            compute(par)
            issue_out(c, par)

    wait_out(0)
    wait_out(1)


def _gather_sum(ids2d, table_n, gamma, beta):
    mesh = plsc.VectorSubcoreMesh(core_axis_name="c", subcore_axis_name="s")
    f = functools.partial(
        pl.kernel,
        out_type=jax.ShapeDtypeStruct((S, D), jnp.float32),
        mesh=mesh,
        compiler_params=pltpu.CompilerParams(
            use_tc_tiling_on_sc=False, needs_layout_passes=False),
        scratch_types=[
            pltpu.VMEM((2, GPC, TPG), jnp.int32),
            pltpu.VMEM((2, CH * L, D), jnp.bfloat16),
            pltpu.VMEM((2, CH, D), jnp.float32),
            pltpu.VMEM((2 * D,), jnp.float32),
            [pltpu.SemaphoreType.DMA, pltpu.SemaphoreType.DMA],
            [pltpu.SemaphoreType.DMA, pltpu.SemaphoreType.DMA],
            [pltpu.SemaphoreType.DMA, pltpu.SemaphoreType.DMA],
        ],
    )(_seg_sum_body)
    return f(ids2d, table_n, gamma, beta)


def kernel(input_ids, table, gamma, beta):
    table_n = _normalize_table(table)
    ids2d = input_ids.reshape(S * L // TPG, TPG)
    out = _gather_sum(ids2d, table_n, gamma, beta)
    return out.reshape(B, F, D)
